# Initial kernel scaffold; baseline (speedup 1.0000x reference)
import jax
import jax.numpy as jnp
from jax.experimental import pallas as pl


def kernel(x, codebook):
    raise NotImplementedError("write your pallas kernel here")


# speedup vs baseline: 1.0646x; 1.0646x over previous
import functools

import jax
import jax.numpy as jnp
from jax import lax
from jax.experimental import pallas as pl
from jax.experimental.pallas import tpu as pltpu
from jax.experimental.pallas import tpu_sc as plsc

NUM_EMB = 8192
DIM = 256
BK = 2048
KB = NUM_EMB // BK
HW = 1024


def _dist_argmin_body(x_ref, cb_ref, x2_ref, w2_ref,
                      minval_ref, idx_ref, loss_ref):
    b = pl.program_id(0)
    k = pl.program_id(1)

    xt = x_ref[0]
    cbb = cb_ref[...]
    cross = lax.dot_general(xt, cbb, (((0,), (1,)), ((), ())),
                            preferred_element_type=jnp.float32)
    x2 = x2_ref[0]
    w2 = w2_ref[...]
    d2 = (x2 + w2) - 2.0 * cross
    dist = jnp.sqrt(jnp.maximum(d2, 0.0))

    lmin = jnp.min(dist, axis=1, keepdims=True)
    ii = lax.broadcasted_iota(jnp.int32, (HW, BK), 1) + k * BK
    big = jnp.int32(2**31 - 1)
    lidx = jnp.min(jnp.where(dist == lmin, ii, big),
                   axis=1, keepdims=True)

    @pl.when(k == 0)
    def _():
        minval_ref[0] = lmin
        idx_ref[0] = lidx

    @pl.when(k > 0)
    def _():
        prev = minval_ref[0]
        better = lmin < prev
        minval_ref[0] = jnp.where(better, lmin, prev)
        idx_ref[0] = jnp.where(better, lidx, idx_ref[0])

    @pl.when(k == KB - 1)
    def _():
        mv = minval_ref[0]
        s = jnp.sum(mv * mv, keepdims=True)

        @pl.when(b == 0)
        def _():
            loss_ref[...] = s

        @pl.when(b > 0)
        def _():
            loss_ref[...] = loss_ref[...] + s


def _dist_argmin(xr, codebook, x2, w2):
    B = xr.shape[0]
    grid = (B, KB)
    out = pl.pallas_call(
        _dist_argmin_body,
        grid=grid,
        in_specs=[
            pl.BlockSpec((1, DIM, HW), lambda b, k: (b, 0, 0)),
            pl.BlockSpec((BK, DIM), lambda b, k: (k, 0)),
            pl.BlockSpec((1, HW, 1), lambda b, k: (b, 0, 0)),
            pl.BlockSpec((1, BK), lambda b, k: (0, k)),
        ],
        out_specs=[
            pl.BlockSpec((1, HW, 1), lambda b, k: (b, 0, 0)),
            pl.BlockSpec((1, HW, 1), lambda b, k: (b, 0, 0)),
            pl.BlockSpec((1, 1), lambda b, k: (0, 0)),
        ],
        out_shape=[
            jax.ShapeDtypeStruct((B, HW, 1), jnp.float32),
            jax.ShapeDtypeStruct((B, HW, 1), jnp.int32),
            jax.ShapeDtypeStruct((1, 1), jnp.float32),
        ],
    )(xr, codebook, x2, w2)
    return out


_SC_WORKERS = 32
_BPW = (8 * HW) // _SC_WORKERS


@functools.lru_cache(maxsize=1)
def _make_sc_gather():
    @functools.partial(
        pl.kernel,
        mesh=plsc.VectorSubcoreMesh(core_axis_name="c", subcore_axis_name="s"),
        out_type=jax.ShapeDtypeStruct((8 * HW, DIM), jnp.float32),
        scratch_types=[
            pltpu.VMEM((_BPW,), jnp.int32),
            pltpu.VMEM((_BPW, DIM), jnp.float32),
            pltpu.SemaphoreType.DMA,
        ],
    )
    def _sc_gather(table_hbm, idx_hbm, out_hbm, idx_v, rows_v, sem):
        wid = lax.axis_index("s") * 2 + lax.axis_index("c")
        base = wid * _BPW
        pltpu.sync_copy(idx_hbm.at[pl.ds(base, _BPW)], idx_v)
        pltpu.async_copy(table_hbm.at[idx_v], rows_v, sem).wait()
        pltpu.sync_copy(rows_v, out_hbm.at[pl.ds(base, _BPW)])

    return _sc_gather


def kernel(x, codebook):
    B, C, H, W = x.shape
    hw = H * W
    xr = x.reshape(B, C, hw)
    xf = jnp.transpose(xr, (0, 2, 1))
    x2 = jnp.sum(xf ** 2, axis=-1, keepdims=True)
    w2 = jnp.sum(codebook ** 2, axis=-1).reshape(1, NUM_EMB)

    minval, idx, loss_sum = _dist_argmin(xr, codebook, x2, w2)

    idx_flat = idx.reshape(B * hw)
    quant = _make_sc_gather()(codebook, idx_flat)

    quant_out = jnp.transpose(quant.reshape(B, hw, C), (0, 2, 1)).reshape(
        B, C, H, W)
    loss = loss_sum[0, 0] / jnp.float32(B * hw * C)
    indices = idx.reshape(B, H, W)
    return quant_out, loss, loss, indices
